# Initial kernel scaffold; baseline (speedup 1.0000x reference)
import jax
import jax.numpy as jnp
from jax.experimental import pallas as pl


def kernel(x, edge_index, W1, b1, W2, b2):
    raise NotImplementedError("write your pallas kernel here")


# speedup vs baseline: 6.8515x; 6.8515x over previous
import functools

import jax
import jax.numpy as jnp
from jax import lax
from jax.experimental import pallas as pl
from jax.experimental.pallas import tpu as pltpu
from jax.experimental.pallas import tpu_sc as plsc

N_NODES = 10000
N_EDGES = 160000
D_IN = 256
D_HID = 256
D_OUT = 64

NP = 10240
EP = 163840
PAD = 10200
NSC = 2
NTILE = 16
ROWS_T = NP // NTILE
CHUNK = 128
NCH_B = EP // NTILE // CHUNK
NCH_W = EP // (NSC * NTILE) // CHUNK
EW = EP // (NSC * NTILE)
DEG_R = NP // 128
DEG_T = DEG_R // NTILE
M_BLK = 256
NB = NP // M_BLK

_MESH = plsc.VectorSubcoreMesh(core_axis_name="c", subcore_axis_name="s",
                               num_cores=NSC, num_subcores=NTILE)


def _deg_body(dst_hbm, zeros_hbm, ident_hbm, out_hbm, idx_v, cnt, ident, acc):
    c = lax.axis_index("c")
    s = lax.axis_index("s")
    w = c * NTILE + s
    pltpu.sync_copy(dst_hbm.at[w], idx_v)
    pltpu.sync_copy(zeros_hbm, cnt)
    pltpu.sync_copy(ident_hbm, ident)
    @pl.when(s < DEG_R // 8)
    def _():
        pltpu.sync_copy(zeros_hbm.at[pl.ds(s * 8, 8)],
                        acc.at[pl.ds(s * 8, 8)])
    plsc.subcore_barrier()

    ones = jnp.full((16,), 1.0, jnp.float32)

    def body(j, carry):
        for l in range(8):
            v = idx_v[j, pl.ds(16 * l, 16)]
            row = jax.lax.shift_right_logical(v, 7)
            col = jax.lax.bitwise_and(v, 127)
            plsc.addupdate_scatter(cnt, [row, col], ones)
        return carry

    lax.fori_loop(0, EW // CHUNK, body, 0)
    pltpu.sync_copy(cnt, acc.at[ident], add=True)
    plsc.subcore_barrier()

    @pl.when(s < DEG_R // 8)
    def _():
        pltpu.sync_copy(acc.at[pl.ds(s * 8, 8)],
                        out_hbm.at[pl.ds(c * DEG_R + s * 8, 8)])


def _make_deg(interpret=False):
    return functools.partial(
        pl.kernel,
        out_type=jax.ShapeDtypeStruct((2 * DEG_R, 128), jnp.float32),
        mesh=_MESH,
        scratch_types=[
            pltpu.VMEM((NCH_W, CHUNK), jnp.int32),
            pltpu.VMEM((DEG_R, 128), jnp.float32),
            pltpu.VMEM((DEG_R,), jnp.int32),
            pltpu.VMEM_SHARED((DEG_R, 128), jnp.float32),
        ],
        compiler_params=pltpu.CompilerParams(needs_layout_passes=False),
        interpret=interpret,
    )(_deg_body)


_deg_kernel = _make_deg()


def _agg_body(col_split, ngrp, ng, y_hbm, src_hbm, dst_hbm, out_hbm,
              idx_s, idx_d, buf0, buf1, acc, sem0, sem1):
    c = lax.axis_index("c")
    s = lax.axis_index("s")
    w = c * NTILE + s
    if col_split:
        ybase = c * NP + s * ROWS_T
    else:
        ybase = s * ROWS_T
    pltpu.sync_copy(y_hbm.at[pl.ds(ybase, ROWS_T)],
                    acc.at[pl.ds(s * ROWS_T, ROWS_T)])
    plsc.subcore_barrier()

    def step(j2, carry):
        g = 2 * j2
        d0 = pltpu.async_copy(y_hbm.at[idx_s.at[g]], buf0, sem0)
        d1 = pltpu.async_copy(y_hbm.at[idx_s.at[g + 1]], buf1, sem1)
        d0.wait()
        pltpu.sync_copy(buf0, acc.at[idx_d.at[g]], add=True)
        d1.wait()
        pltpu.sync_copy(buf1, acc.at[idx_d.at[g + 1]], add=True)
        return carry

    for grp in range(ngrp):
        pltpu.sync_copy(src_hbm.at[w * ngrp + grp], idx_s)
        dbase = (s if col_split else w) * ngrp + grp
        pltpu.sync_copy(dst_hbm.at[dbase], idx_d)
        lax.fori_loop(0, ng // 2, step, 0)
    plsc.subcore_barrier()
    pltpu.sync_copy(acc.at[pl.ds(s * ROWS_T, ROWS_T)],
                    out_hbm.at[pl.ds(c * NP + s * ROWS_T, ROWS_T)])


def _make_agg(col_split, ngrp, ng, interpret=False):
    return functools.partial(
        pl.kernel,
        out_type=jax.ShapeDtypeStruct((2 * NP, 128), jnp.float32),
        mesh=_MESH,
        scratch_types=[
            pltpu.VMEM((ng, CHUNK), jnp.int32),
            pltpu.VMEM((ng, CHUNK), jnp.int32),
            pltpu.VMEM((CHUNK, 128), jnp.float32),
            pltpu.VMEM((CHUNK, 128), jnp.float32),
            pltpu.VMEM_SHARED((NP, 128), jnp.float32),
            pltpu.SemaphoreType.DMA,
            pltpu.SemaphoreType.DMA,
        ],
        interpret=interpret,
    )(functools.partial(_agg_body, col_split, ngrp, ng))


NGRP1 = 2
_agg1_kernel = _make_agg(True, NGRP1, NCH_B // NGRP1)
_agg2_kernel = _make_agg(False, 1, NCH_W)


def _dis(dA_ref, dB_ref):
    return lax.rsqrt(1.0 + dA_ref[...] + dB_ref[...])


def _mm1_body(x_ref, w1_ref, dA_ref, dB_ref, o_ref):
    y = jnp.dot(x_ref[...], w1_ref[...], preferred_element_type=jnp.float32)
    o_ref[...] = _dis(dA_ref, dB_ref) * y


_mm1 = pl.pallas_call(
    _mm1_body,
    grid=(2, NB),
    in_specs=[
        pl.BlockSpec((M_BLK, D_IN), lambda c, i: (i, 0)),
        pl.BlockSpec((D_IN, D_HID // 2), lambda c, i: (0, c)),
        pl.BlockSpec((M_BLK, 1), lambda c, i: (i, 0)),
        pl.BlockSpec((M_BLK, 1), lambda c, i: (i + NB, 0)),
    ],
    out_specs=pl.BlockSpec((M_BLK, D_HID // 2), lambda c, i: (c * NB + i, 0)),
    out_shape=jax.ShapeDtypeStruct((2 * NP, D_HID // 2), jnp.float32),
)


def _mm2_body(aA_ref, aB_ref, dA_ref, dB_ref, b1_ref, w2_ref, o_ref):
    dis = _dis(dA_ref, dB_ref)
    hA = jnp.maximum(dis * aA_ref[...] + b1_ref[0:1, : D_HID // 2], 0.0)
    hB = jnp.maximum(dis * aB_ref[...] + b1_ref[0:1, D_HID // 2:], 0.0)
    y = (jnp.dot(hA, w2_ref[: D_HID // 2], preferred_element_type=jnp.float32)
         + jnp.dot(hB, w2_ref[D_HID // 2:], preferred_element_type=jnp.float32))
    o_ref[...] = jnp.pad(dis * y, ((0, 0), (0, 128 - D_OUT)))


_mm2 = pl.pallas_call(
    _mm2_body,
    grid=(NB,),
    in_specs=[
        pl.BlockSpec((M_BLK, D_HID // 2), lambda i: (i, 0)),
        pl.BlockSpec((M_BLK, D_HID // 2), lambda i: (i + NB, 0)),
        pl.BlockSpec((M_BLK, 1), lambda i: (i, 0)),
        pl.BlockSpec((M_BLK, 1), lambda i: (i + NB, 0)),
        pl.BlockSpec((1, D_HID), lambda i: (0, 0)),
        pl.BlockSpec((D_HID, D_OUT), lambda i: (0, 0)),
    ],
    out_specs=pl.BlockSpec((M_BLK, 128), lambda i: (i, 0)),
    out_shape=jax.ShapeDtypeStruct((NP, 128), jnp.float32),
)


def _fin_body(aA_ref, aB_ref, y2_ref, dA_ref, dB_ref, b2_ref, o_ref):
    dis = _dis(dA_ref, dB_ref)
    agg = (aA_ref[:, :D_OUT] + aB_ref[:, :D_OUT] - y2_ref[:, :D_OUT])
    o_ref[...] = dis * agg + b2_ref[0:1, :]


_fin = pl.pallas_call(
    _fin_body,
    grid=(NB,),
    in_specs=[
        pl.BlockSpec((M_BLK, 128), lambda i: (i, 0)),
        pl.BlockSpec((M_BLK, 128), lambda i: (i + NB, 0)),
        pl.BlockSpec((M_BLK, 128), lambda i: (i, 0)),
        pl.BlockSpec((M_BLK, 1), lambda i: (i, 0)),
        pl.BlockSpec((M_BLK, 1), lambda i: (i + NB, 0)),
        pl.BlockSpec((1, D_OUT), lambda i: (0, 0)),
    ],
    out_specs=pl.BlockSpec((M_BLK, D_OUT), lambda i: (i, 0)),
    out_shape=jax.ShapeDtypeStruct((NP, D_OUT), jnp.float32),
)


def kernel(x, edge_index, W1, b1, W2, b2):
    src = edge_index[0].astype(jnp.int32)
    dst = edge_index[1].astype(jnp.int32)
    epad = jnp.full((EP - N_EDGES,), PAD, jnp.int32)
    srcp = jnp.concatenate([src, epad])
    dstp = jnp.concatenate([dst, epad])
    x_pad = jnp.concatenate(
        [x, jnp.zeros((NP - N_NODES, D_IN), jnp.float32)])

    dst_deg = dstp.reshape(NSC * NTILE, NCH_W, CHUNK)
    dst_w = dstp.reshape(NSC * NTILE, NCH_W, CHUNK)
    src_w = srcp.reshape(NSC * NTILE, NCH_W, CHUNK)
    dst_b = dstp.reshape(NTILE * NGRP1, NCH_B // NGRP1, CHUNK)
    src_b = jnp.stack([srcp, srcp + NP]).reshape(
        NSC * NTILE * NGRP1, NCH_B // NGRP1, CHUNK)

    zeros_slab = jnp.zeros((DEG_R, 128), jnp.float32)
    ident = jnp.arange(DEG_R, dtype=jnp.int32)

    deg = _deg_kernel(dst_deg, zeros_slab, ident)
    degr = deg.reshape(2 * NP, 1)
    y1 = _mm1(x_pad, W1, degr, degr)
    acc1 = _agg1_kernel(y1, src_b, dst_b)
    y2 = _mm2(acc1, acc1, degr, degr, b1.reshape(1, D_HID), W2)
    acc2 = _agg2_kernel(y2, src_w, dst_w)
    out = _fin(acc2, acc2, y2, degr, degr, b2.reshape(1, D_OUT))
    return out[:N_NODES]


# speedup vs baseline: 6.8819x; 1.0044x over previous
import functools

import jax
import jax.numpy as jnp
from jax import lax
from jax.experimental import pallas as pl
from jax.experimental.pallas import tpu as pltpu
from jax.experimental.pallas import tpu_sc as plsc

N_NODES = 10000
N_EDGES = 160000
D_IN = 256
D_HID = 256
D_OUT = 64

NP = 10240
EP = 163840
PAD = 10200
NSC = 2
NTILE = 16
ROWS_T = NP // NTILE
CHUNK = 128
NCH_B = EP // NTILE // CHUNK
NCH_W = EP // (NSC * NTILE) // CHUNK
EW = EP // (NSC * NTILE)
DEG_R = NP // 128
DEG_T = DEG_R // NTILE
M_BLK = 256
NB = NP // M_BLK

_MESH = plsc.VectorSubcoreMesh(core_axis_name="c", subcore_axis_name="s",
                               num_cores=NSC, num_subcores=NTILE)


def _deg_body(dst_hbm, zeros_hbm, ident_hbm, out_hbm, idx_v, cnt, ident, acc):
    c = lax.axis_index("c")
    s = lax.axis_index("s")
    w = c * NTILE + s
    pltpu.sync_copy(dst_hbm.at[w], idx_v)
    pltpu.sync_copy(zeros_hbm, cnt)
    pltpu.sync_copy(ident_hbm, ident)
    @pl.when(s < DEG_R // 8)
    def _():
        pltpu.sync_copy(zeros_hbm.at[pl.ds(s * 8, 8)],
                        acc.at[pl.ds(s * 8, 8)])
    plsc.subcore_barrier()

    ones = jnp.full((16,), 1.0, jnp.float32)

    def body(j, carry):
        for l in range(8):
            v = idx_v[j, pl.ds(16 * l, 16)]
            row = jax.lax.shift_right_logical(v, 7)
            col = jax.lax.bitwise_and(v, 127)
            plsc.addupdate_scatter(cnt, [row, col], ones)
        return carry

    lax.fori_loop(0, EW // CHUNK, body, 0)
    pltpu.sync_copy(cnt, acc.at[ident], add=True)
    plsc.subcore_barrier()

    @pl.when(s < DEG_R // 8)
    def _():
        pltpu.sync_copy(acc.at[pl.ds(s * 8, 8)],
                        out_hbm.at[pl.ds(c * DEG_R + s * 8, 8)])


def _make_deg(interpret=False):
    return functools.partial(
        pl.kernel,
        out_type=jax.ShapeDtypeStruct((2 * DEG_R, 128), jnp.float32),
        mesh=_MESH,
        scratch_types=[
            pltpu.VMEM((NCH_W, CHUNK), jnp.int32),
            pltpu.VMEM((DEG_R, 128), jnp.float32),
            pltpu.VMEM((DEG_R,), jnp.int32),
            pltpu.VMEM_SHARED((DEG_R, 128), jnp.float32),
        ],
        compiler_params=pltpu.CompilerParams(needs_layout_passes=False),
        interpret=interpret,
    )(_deg_body)


_deg_kernel = _make_deg()


def _agg_body(col_split, ngrp, ng, y_hbm, src_hbm, dst_hbm, out_hbm,
              idx_s, idx_d, buf0, buf1, acc, sem0, sem1, sem_s0, sem_s1):
    c = lax.axis_index("c")
    s = lax.axis_index("s")
    w = c * NTILE + s
    if col_split:
        ybase = c * NP + s * ROWS_T
    else:
        ybase = s * ROWS_T
    pltpu.sync_copy(y_hbm.at[pl.ds(ybase, ROWS_T)],
                    acc.at[pl.ds(s * ROWS_T, ROWS_T)])
    plsc.subcore_barrier()

    def drain(g0, g1):
        pltpu.make_async_copy(buf0, acc.at[idx_d.at[g0]], sem_s0).wait()
        pltpu.make_async_copy(buf1, acc.at[idx_d.at[g1]], sem_s1).wait()

    def step(j2, carry):
        g = 2 * j2
        @pl.when(j2 > 0)
        def _():
            drain(g - 2, g - 1)
        d0 = pltpu.async_copy(y_hbm.at[idx_s.at[g]], buf0, sem0)
        d1 = pltpu.async_copy(y_hbm.at[idx_s.at[g + 1]], buf1, sem1)
        d0.wait()
        pltpu.async_copy(buf0, acc.at[idx_d.at[g]], sem_s0, add=True)
        d1.wait()
        pltpu.async_copy(buf1, acc.at[idx_d.at[g + 1]], sem_s1, add=True)
        return carry

    for grp in range(ngrp):
        if grp > 0:
            drain(ng - 2, ng - 1)
        pltpu.sync_copy(src_hbm.at[w * ngrp + grp], idx_s)
        dbase = (s if col_split else w) * ngrp + grp
        pltpu.sync_copy(dst_hbm.at[dbase], idx_d)
        lax.fori_loop(0, ng // 2, step, 0)
    drain(ng - 2, ng - 1)
    plsc.subcore_barrier()
    pltpu.sync_copy(acc.at[pl.ds(s * ROWS_T, ROWS_T)],
                    out_hbm.at[pl.ds(c * NP + s * ROWS_T, ROWS_T)])


def _make_agg(col_split, ngrp, ng, interpret=False):
    return functools.partial(
        pl.kernel,
        out_type=jax.ShapeDtypeStruct((2 * NP, 128), jnp.float32),
        mesh=_MESH,
        scratch_types=[
            pltpu.VMEM((ng, CHUNK), jnp.int32),
            pltpu.VMEM((ng, CHUNK), jnp.int32),
            pltpu.VMEM((CHUNK, 128), jnp.float32),
            pltpu.VMEM((CHUNK, 128), jnp.float32),
            pltpu.VMEM_SHARED((NP, 128), jnp.float32),
            pltpu.SemaphoreType.DMA,
            pltpu.SemaphoreType.DMA,
            pltpu.SemaphoreType.DMA,
            pltpu.SemaphoreType.DMA,
        ],
        interpret=interpret,
    )(functools.partial(_agg_body, col_split, ngrp, ng))


NGRP1 = 2
_agg1_kernel = _make_agg(True, NGRP1, NCH_B // NGRP1)
_agg2_kernel = _make_agg(False, 1, NCH_W)


def _dis(dA_ref, dB_ref):
    return lax.rsqrt(1.0 + dA_ref[...] + dB_ref[...])


def _mm1_body(x_ref, w1_ref, dA_ref, dB_ref, o_ref):
    y = jnp.dot(x_ref[...], w1_ref[...], preferred_element_type=jnp.float32)
    o_ref[...] = _dis(dA_ref, dB_ref) * y


_mm1 = pl.pallas_call(
    _mm1_body,
    grid=(2, NB),
    in_specs=[
        pl.BlockSpec((M_BLK, D_IN), lambda c, i: (i, 0)),
        pl.BlockSpec((D_IN, D_HID // 2), lambda c, i: (0, c)),
        pl.BlockSpec((M_BLK, 1), lambda c, i: (i, 0)),
        pl.BlockSpec((M_BLK, 1), lambda c, i: (i + NB, 0)),
    ],
    out_specs=pl.BlockSpec((M_BLK, D_HID // 2), lambda c, i: (c * NB + i, 0)),
    out_shape=jax.ShapeDtypeStruct((2 * NP, D_HID // 2), jnp.float32),
)


def _mm2_body(aA_ref, aB_ref, dA_ref, dB_ref, b1_ref, w2_ref, o_ref):
    dis = _dis(dA_ref, dB_ref)
    hA = jnp.maximum(dis * aA_ref[...] + b1_ref[0:1, : D_HID // 2], 0.0)
    hB = jnp.maximum(dis * aB_ref[...] + b1_ref[0:1, D_HID // 2:], 0.0)
    y = (jnp.dot(hA, w2_ref[: D_HID // 2], preferred_element_type=jnp.float32)
         + jnp.dot(hB, w2_ref[D_HID // 2:], preferred_element_type=jnp.float32))
    o_ref[...] = jnp.pad(dis * y, ((0, 0), (0, 128 - D_OUT)))


_mm2 = pl.pallas_call(
    _mm2_body,
    grid=(NB,),
    in_specs=[
        pl.BlockSpec((M_BLK, D_HID // 2), lambda i: (i, 0)),
        pl.BlockSpec((M_BLK, D_HID // 2), lambda i: (i + NB, 0)),
        pl.BlockSpec((M_BLK, 1), lambda i: (i, 0)),
        pl.BlockSpec((M_BLK, 1), lambda i: (i + NB, 0)),
        pl.BlockSpec((1, D_HID), lambda i: (0, 0)),
        pl.BlockSpec((D_HID, D_OUT), lambda i: (0, 0)),
    ],
    out_specs=pl.BlockSpec((M_BLK, 128), lambda i: (i, 0)),
    out_shape=jax.ShapeDtypeStruct((NP, 128), jnp.float32),
)


def _fin_body(aA_ref, aB_ref, y2_ref, dA_ref, dB_ref, b2_ref, o_ref):
    dis = _dis(dA_ref, dB_ref)
    agg = (aA_ref[:, :D_OUT] + aB_ref[:, :D_OUT] - y2_ref[:, :D_OUT])
    o_ref[...] = dis * agg + b2_ref[0:1, :]


_fin = pl.pallas_call(
    _fin_body,
    grid=(NB,),
    in_specs=[
        pl.BlockSpec((M_BLK, 128), lambda i: (i, 0)),
        pl.BlockSpec((M_BLK, 128), lambda i: (i + NB, 0)),
        pl.BlockSpec((M_BLK, 128), lambda i: (i, 0)),
        pl.BlockSpec((M_BLK, 1), lambda i: (i, 0)),
        pl.BlockSpec((M_BLK, 1), lambda i: (i + NB, 0)),
        pl.BlockSpec((1, D_OUT), lambda i: (0, 0)),
    ],
    out_specs=pl.BlockSpec((M_BLK, D_OUT), lambda i: (i, 0)),
    out_shape=jax.ShapeDtypeStruct((NP, D_OUT), jnp.float32),
)


def kernel(x, edge_index, W1, b1, W2, b2):
    src = edge_index[0].astype(jnp.int32)
    dst = edge_index[1].astype(jnp.int32)
    epad = jnp.full((EP - N_EDGES,), PAD, jnp.int32)
    srcp = jnp.concatenate([src, epad])
    dstp = jnp.concatenate([dst, epad])
    x_pad = jnp.concatenate(
        [x, jnp.zeros((NP - N_NODES, D_IN), jnp.float32)])

    dst_deg = dstp.reshape(NSC * NTILE, NCH_W, CHUNK)
    dst_w = dstp.reshape(NSC * NTILE, NCH_W, CHUNK)
    src_w = srcp.reshape(NSC * NTILE, NCH_W, CHUNK)
    dst_b = dstp.reshape(NTILE * NGRP1, NCH_B // NGRP1, CHUNK)
    src_b = jnp.stack([srcp, srcp + NP]).reshape(
        NSC * NTILE * NGRP1, NCH_B // NGRP1, CHUNK)

    zeros_slab = jnp.zeros((DEG_R, 128), jnp.float32)
    ident = jnp.arange(DEG_R, dtype=jnp.int32)

    deg = _deg_kernel(dst_deg, zeros_slab, ident)
    degr = deg.reshape(2 * NP, 1)
    y1 = _mm1(x_pad, W1, degr, degr)
    acc1 = _agg1_kernel(y1, src_b, dst_b)
    y2 = _mm2(acc1, acc1, degr, degr, b1.reshape(1, D_HID), W2)
    acc2 = _agg2_kernel(y2, src_w, dst_w)
    out = _fin(acc2, acc2, y2, degr, degr, b2.reshape(1, D_OUT))
    return out[:N_NODES]


# speedup vs baseline: 6.9848x; 1.0149x over previous
import functools

import jax
import jax.numpy as jnp
from jax import lax
from jax.experimental import pallas as pl
from jax.experimental.pallas import tpu as pltpu
from jax.experimental.pallas import tpu_sc as plsc

N_NODES = 10000
N_EDGES = 160000
D_IN = 256
D_HID = 256
D_OUT = 64

NP = 10240
EP = 163840
PAD = 10200
NSC = 2
NTILE = 16
ROWS_T = NP // NTILE
CHUNK = 128
NCH_B = EP // NTILE // CHUNK
NCH_W = EP // (NSC * NTILE) // CHUNK
EW = EP // (NSC * NTILE)
DEG_R = NP // 128
DEG_T = DEG_R // NTILE
M_BLK = 256
NB = NP // M_BLK
NG_SKEW = 16

_MESH = plsc.VectorSubcoreMesh(core_axis_name="c", subcore_axis_name="s",
                               num_cores=NSC, num_subcores=NTILE)


def _deg_body(dst_hbm, zeros_hbm, ident_hbm, out_hbm, idx_v, cnt, ident, acc):
    c = lax.axis_index("c")
    s = lax.axis_index("s")
    w = c * NTILE + s
    pltpu.sync_copy(dst_hbm.at[w], idx_v)
    pltpu.sync_copy(zeros_hbm, cnt)
    pltpu.sync_copy(ident_hbm, ident)
    @pl.when(s < DEG_R // 8)
    def _():
        pltpu.sync_copy(zeros_hbm.at[pl.ds(s * 8, 8)],
                        acc.at[pl.ds(s * 8, 8)])
    plsc.subcore_barrier()

    ones = jnp.full((16,), 1.0, jnp.float32)

    def body(j, carry):
        for l in range(8):
            v = idx_v[j, pl.ds(16 * l, 16)]
            row = jax.lax.shift_right_logical(v, 7)
            col = jax.lax.bitwise_and(v, 127)
            plsc.addupdate_scatter(cnt, [row, col], ones)
        return carry

    lax.fori_loop(0, EW // CHUNK, body, 0)
    pltpu.sync_copy(cnt, acc.at[ident], add=True)
    plsc.subcore_barrier()

    @pl.when(s < DEG_R // 8)
    def _():
        pltpu.sync_copy(acc.at[pl.ds(s * 8, 8)],
                        out_hbm.at[pl.ds(c * DEG_R + s * 8, 8)])


def _make_deg(interpret=False):
    return functools.partial(
        pl.kernel,
        out_type=jax.ShapeDtypeStruct((2 * DEG_R, 128), jnp.float32),
        mesh=_MESH,
        scratch_types=[
            pltpu.VMEM((NCH_W, CHUNK), jnp.int32),
            pltpu.VMEM((DEG_R, 128), jnp.float32),
            pltpu.VMEM((DEG_R,), jnp.int32),
            pltpu.VMEM_SHARED((DEG_R, 128), jnp.float32),
        ],
        compiler_params=pltpu.CompilerParams(needs_layout_passes=False),
        interpret=interpret,
    )(_deg_body)


_deg_kernel = _make_deg()


def _agg_body(col_split, ngrp, ng, y_hbm, src_hbm, dst_hbm, ini_hbm,
              out_hbm, idx_s, idx_d, buf0, buf1, acc,
              sem0, sem1, sem_s0, sem_s1):
    c = lax.axis_index("c")
    s = lax.axis_index("s")
    w = c * NTILE + s
    y_tab = y_hbm
    if col_split:
        for j in range(ROWS_T // CHUNK):
            pltpu.sync_copy(ini_hbm.at[w * (ROWS_T // CHUNK) + j],
                            idx_s.at[0])
            pltpu.sync_copy(y_hbm.at[idx_s.at[0]], buf0)
            pltpu.sync_copy(buf0, acc.at[pl.ds(s * ROWS_T + j * CHUNK,
                                               CHUNK)])
    else:
        pltpu.sync_copy(y_tab.at[pl.ds(s * ROWS_T, ROWS_T)],
                        acc.at[pl.ds(s * ROWS_T, ROWS_T)])
    plsc.subcore_barrier()

    def drain(g0, g1):
        pltpu.make_async_copy(buf0, acc.at[idx_d.at[g0]], sem_s0).wait()
        pltpu.make_async_copy(buf1, acc.at[idx_d.at[g1]], sem_s1).wait()

    def step(j2, carry):
        g = 2 * j2
        @pl.when(j2 > 0)
        def _():
            drain(g - 2, g - 1)
        d0 = pltpu.async_copy(y_tab.at[idx_s.at[g]], buf0, sem0)
        d1 = pltpu.async_copy(y_tab.at[idx_s.at[g + 1]], buf1, sem1)
        d0.wait()
        pltpu.async_copy(buf0, acc.at[idx_d.at[g]], sem_s0, add=True)
        d1.wait()
        pltpu.async_copy(buf1, acc.at[idx_d.at[g + 1]], sem_s1, add=True)
        return carry

    if col_split:
        for grp in range(ngrp):
            if grp > 0:
                drain(ng - 2, ng - 1)
            pltpu.sync_copy(src_hbm.at[w * ngrp + grp], idx_s)
            pltpu.sync_copy(dst_hbm.at[s * ngrp + grp], idx_d)
            lax.fori_loop(0, ng // 2, step, 0)
    else:
        ng0, ng1 = ng - NG_SKEW, ng + NG_SKEW
        n_w = jnp.where(c == 0, ng0, ng1)
        start = pl.multiple_of(
            jnp.where(c == 0, s * ng0, NTILE * ng0 + s * ng1), 8)
        pltpu.sync_copy(src_hbm.at[pl.ds(start, ng1)], idx_s)
        pltpu.sync_copy(dst_hbm.at[pl.ds(start, ng1)], idx_d)
        lax.fori_loop(0, n_w // 2, step, 0)
    drain(ng - 2, ng - 1)
    plsc.subcore_barrier()
    pltpu.sync_copy(acc.at[pl.ds(s * ROWS_T, ROWS_T)],
                    out_hbm.at[pl.ds(c * NP + s * ROWS_T, ROWS_T)])


def _make_agg(col_split, ngrp, ng, interpret=False):
    return functools.partial(
        pl.kernel,
        out_type=jax.ShapeDtypeStruct((2 * NP, 128), jnp.float32),
        mesh=_MESH,
        scratch_types=[
            pltpu.VMEM((ng if col_split else ng + NG_SKEW, CHUNK), jnp.int32),
            pltpu.VMEM((ng if col_split else ng + NG_SKEW, CHUNK), jnp.int32),
            pltpu.VMEM((CHUNK, 128), jnp.float32),
            pltpu.VMEM((CHUNK, 128), jnp.float32),
            pltpu.VMEM_SHARED((NP, 128), jnp.float32),
            pltpu.SemaphoreType.DMA,
            pltpu.SemaphoreType.DMA,
            pltpu.SemaphoreType.DMA,
            pltpu.SemaphoreType.DMA,
        ],
        interpret=interpret,
    )(functools.partial(_agg_body, col_split, ngrp, ng))


NGRP1 = 2
_agg1_kernel = _make_agg(True, NGRP1, NCH_B // NGRP1)
_agg2_kernel = _make_agg(False, 1, NCH_W)


def _dis(dA_ref, dB_ref):
    return lax.rsqrt(1.0 + dA_ref[...] + dB_ref[...])


def _mm1_body(x_ref, w1_ref, dA_ref, dB_ref, o_ref):
    y = jnp.dot(x_ref[...], w1_ref[...], preferred_element_type=jnp.float32)
    o_ref[...] = _dis(dA_ref, dB_ref) * y


_mm1 = pl.pallas_call(
    _mm1_body,
    grid=(NB,),
    in_specs=[
        pl.BlockSpec((M_BLK, D_IN), lambda i: (i, 0)),
        pl.BlockSpec((D_IN, D_HID), lambda i: (0, 0)),
        pl.BlockSpec((M_BLK, 1), lambda i: (i, 0)),
        pl.BlockSpec((M_BLK, 1), lambda i: (i + NB, 0)),
    ],
    out_specs=pl.BlockSpec((M_BLK, D_HID), lambda i: (i, 0)),
    out_shape=jax.ShapeDtypeStruct((NP, D_HID), jnp.float32),
)


def _mm2_body(aA_ref, aB_ref, dA_ref, dB_ref, b1_ref, w2_ref, o_ref):
    dis = _dis(dA_ref, dB_ref)
    hA = jnp.maximum(dis * aA_ref[...] + b1_ref[0:1, : D_HID // 2], 0.0)
    hB = jnp.maximum(dis * aB_ref[...] + b1_ref[0:1, D_HID // 2:], 0.0)
    y = (jnp.dot(hA, w2_ref[: D_HID // 2], preferred_element_type=jnp.float32)
         + jnp.dot(hB, w2_ref[D_HID // 2:], preferred_element_type=jnp.float32))
    o_ref[...] = jnp.pad(dis * y, ((0, 0), (0, 128 - D_OUT)))


_mm2 = pl.pallas_call(
    _mm2_body,
    grid=(NB,),
    in_specs=[
        pl.BlockSpec((M_BLK, D_HID // 2), lambda i: (i, 0)),
        pl.BlockSpec((M_BLK, D_HID // 2), lambda i: (i + NB, 0)),
        pl.BlockSpec((M_BLK, 1), lambda i: (i, 0)),
        pl.BlockSpec((M_BLK, 1), lambda i: (i + NB, 0)),
        pl.BlockSpec((1, D_HID), lambda i: (0, 0)),
        pl.BlockSpec((D_HID, D_OUT), lambda i: (0, 0)),
    ],
    out_specs=pl.BlockSpec((M_BLK, 128), lambda i: (i, 0)),
    out_shape=jax.ShapeDtypeStruct((NP, 128), jnp.float32),
)


def _fin_body(aA_ref, aB_ref, y2_ref, dA_ref, dB_ref, b2_ref, o_ref):
    dis = _dis(dA_ref, dB_ref)
    agg = (aA_ref[:, :D_OUT] + aB_ref[:, :D_OUT] - y2_ref[:, :D_OUT])
    o_ref[...] = dis * agg + b2_ref[0:1, :]


_fin = pl.pallas_call(
    _fin_body,
    grid=(NB,),
    in_specs=[
        pl.BlockSpec((M_BLK, 128), lambda i: (i, 0)),
        pl.BlockSpec((M_BLK, 128), lambda i: (i + NB, 0)),
        pl.BlockSpec((M_BLK, 128), lambda i: (i, 0)),
        pl.BlockSpec((M_BLK, 1), lambda i: (i, 0)),
        pl.BlockSpec((M_BLK, 1), lambda i: (i + NB, 0)),
        pl.BlockSpec((1, D_OUT), lambda i: (0, 0)),
    ],
    out_specs=pl.BlockSpec((M_BLK, D_OUT), lambda i: (i, 0)),
    out_shape=jax.ShapeDtypeStruct((NP, D_OUT), jnp.float32),
)


def kernel(x, edge_index, W1, b1, W2, b2):
    src = edge_index[0].astype(jnp.int32)
    dst = edge_index[1].astype(jnp.int32)
    epad = jnp.full((EP - N_EDGES,), PAD, jnp.int32)
    srcp = jnp.concatenate([src, epad])
    dstp = jnp.concatenate([dst, epad])
    x_pad = jnp.concatenate(
        [x, jnp.zeros((NP - N_NODES, D_IN), jnp.float32)])

    dst_w = dstp.reshape(NSC * NTILE, NCH_W, CHUNK)
    src_w = srcp.reshape(NSC * NTILE, NCH_W, CHUNK)
    dst_f = dstp.reshape(EP // CHUNK, CHUNK)
    src_f = srcp.reshape(EP // CHUNK, CHUNK)
    src_b = jnp.stack([2 * srcp, 2 * srcp + 1]).reshape(
        NSC * NTILE * NGRP1, NCH_B // NGRP1, CHUNK)
    dst_b = dstp.reshape(NTILE * NGRP1, NCH_B // NGRP1, CHUNK)
    ar = jnp.arange(NP, dtype=jnp.int32)
    ini = jnp.stack([2 * ar, 2 * ar + 1]).reshape(
        NSC * NTILE * (ROWS_T // CHUNK), CHUNK)
    dummy_ini = jnp.zeros((8, CHUNK), jnp.int32)

    zeros_slab = jnp.zeros((DEG_R, 128), jnp.float32)
    ident = jnp.arange(DEG_R, dtype=jnp.int32)

    deg = _deg_kernel(dst_deg, zeros_slab, ident)
    degr = deg.reshape(2 * NP, 1)
    y1 = _mm1(x_pad, W1, degr, degr)
    acc1 = _agg1_kernel(y1, src_b, dst_b)
    y2 = _mm2(acc1, acc1, degr, degr, b1.reshape(1, D_HID), W2)
    acc2 = _agg2_kernel(y2, src_f, dst_f, dummy_ini)
    out = _fin(acc2, acc2, y2, degr, degr, b2.reshape(1, D_OUT))
    return out[:N_NODES]


# speedup vs baseline: 7.0051x; 1.0029x over previous
import functools

import jax
import jax.numpy as jnp
from jax import lax
from jax.experimental import pallas as pl
from jax.experimental.pallas import tpu as pltpu
from jax.experimental.pallas import tpu_sc as plsc

N_NODES = 10000
N_EDGES = 160000
D_IN = 256
D_HID = 256
D_OUT = 64

NP = 10240
EP = 163840
PAD = 10200
NSC = 2
NTILE = 16
ROWS_T = NP // NTILE
CHUNK = 128
NCH_B = EP // NTILE // CHUNK
NCH_W = EP // (NSC * NTILE) // CHUNK
EW = EP // (NSC * NTILE)
DEG_R = NP // 128
DEG_T = DEG_R // NTILE
M_BLK = 256
NB = NP // M_BLK
NG_SKEW = 16

_MESH = plsc.VectorSubcoreMesh(core_axis_name="c", subcore_axis_name="s",
                               num_cores=NSC, num_subcores=NTILE)


def _deg_body(dst_hbm, zeros_hbm, ident_hbm, out_hbm, idx_v, cnt, ident, acc):
    c = lax.axis_index("c")
    s = lax.axis_index("s")
    w = c * NTILE + s
    pltpu.sync_copy(dst_hbm.at[w], idx_v)
    pltpu.sync_copy(zeros_hbm, cnt)
    pltpu.sync_copy(ident_hbm, ident)
    @pl.when(s < DEG_R // 8)
    def _():
        pltpu.sync_copy(zeros_hbm.at[pl.ds(s * 8, 8)],
                        acc.at[pl.ds(s * 8, 8)])
    plsc.subcore_barrier()

    ones = jnp.full((16,), 1.0, jnp.float32)

    def body(j, carry):
        for l in range(8):
            v = idx_v[j, pl.ds(16 * l, 16)]
            row = jax.lax.shift_right_logical(v, 7)
            col = jax.lax.bitwise_and(v, 127)
            plsc.addupdate_scatter(cnt, [row, col], ones)
        return carry

    lax.fori_loop(0, EW // CHUNK, body, 0)
    pltpu.sync_copy(cnt, acc.at[ident], add=True)
    plsc.subcore_barrier()

    @pl.when(s < DEG_R // 8)
    def _():
        pltpu.sync_copy(acc.at[pl.ds(s * 8, 8)],
                        out_hbm.at[pl.ds(c * DEG_R + s * 8, 8)])


def _make_deg():
    return functools.partial(
        pl.kernel,
        out_type=jax.ShapeDtypeStruct((2 * DEG_R, 128), jnp.float32),
        mesh=_MESH,
        scratch_types=[
            pltpu.VMEM((NCH_W, CHUNK), jnp.int32),
            pltpu.VMEM((DEG_R, 128), jnp.float32),
            pltpu.VMEM((DEG_R,), jnp.int32),
            pltpu.VMEM_SHARED((DEG_R, 128), jnp.float32),
        ],
        compiler_params=pltpu.CompilerParams(needs_layout_passes=False),
    )(_deg_body)


_deg_kernel = _make_deg()


def _agg_body(col_split, ngrp, ng, y_hbm, src_hbm, dst_hbm, out_hbm,
              idx_s, idx_d, buf0, buf1, acc, sem0, sem1, sem_s0, sem_s1):
    c = lax.axis_index("c")
    s = lax.axis_index("s")
    w = c * NTILE + s
    if col_split:
        y_tab = y_hbm.at[c]
    else:
        y_tab = y_hbm
    pltpu.sync_copy(y_tab.at[pl.ds(s * ROWS_T, ROWS_T)],
                    acc.at[pl.ds(s * ROWS_T, ROWS_T)])
    plsc.subcore_barrier()

    def drain(g0, g1):
        pltpu.make_async_copy(buf0, acc.at[idx_d.at[g0]], sem_s0).wait()
        pltpu.make_async_copy(buf1, acc.at[idx_d.at[g1]], sem_s1).wait()

    def step(j2, carry):
        g = 2 * j2
        @pl.when(j2 > 0)
        def _():
            drain(g - 2, g - 1)
        d0 = pltpu.async_copy(y_tab.at[idx_s.at[g]], buf0, sem0)
        d1 = pltpu.async_copy(y_tab.at[idx_s.at[g + 1]], buf1, sem1)
        d0.wait()
        pltpu.async_copy(buf0, acc.at[idx_d.at[g]], sem_s0, add=True)
        d1.wait()
        pltpu.async_copy(buf1, acc.at[idx_d.at[g + 1]], sem_s1, add=True)
        return carry

    if col_split:
        for grp in range(ngrp):
            if grp > 0:
                drain(ng - 2, ng - 1)
            ibase = s * ngrp + grp
            pltpu.sync_copy(src_hbm.at[ibase], idx_s)
            pltpu.sync_copy(dst_hbm.at[ibase], idx_d)
            lax.fori_loop(0, ng // 2, step, 0)
    else:
        ng0, ng1 = ng - NG_SKEW, ng + NG_SKEW
        n_w = jnp.where(c == 0, ng0, ng1)
        start = pl.multiple_of(
            jnp.where(c == 0, s * ng0, NTILE * ng0 + s * ng1), 8)
        pltpu.sync_copy(src_hbm.at[pl.ds(start, ng1)], idx_s)
        pltpu.sync_copy(dst_hbm.at[pl.ds(start, ng1)], idx_d)
        lax.fori_loop(0, n_w // 2, step, 0)
    drain(ng - 2, ng - 1)
    plsc.subcore_barrier()
    pltpu.sync_copy(acc.at[pl.ds(s * ROWS_T, ROWS_T)],
                    out_hbm.at[pl.ds(c * NP + s * ROWS_T, ROWS_T)])


def _make_agg(col_split, ngrp, ng):
    return functools.partial(
        pl.kernel,
        out_type=jax.ShapeDtypeStruct((2 * NP, 128), jnp.float32),
        mesh=_MESH,
        scratch_types=[
            pltpu.VMEM((ng if col_split else ng + NG_SKEW, CHUNK), jnp.int32),
            pltpu.VMEM((ng if col_split else ng + NG_SKEW, CHUNK), jnp.int32),
            pltpu.VMEM((CHUNK, 128), jnp.float32),
            pltpu.VMEM((CHUNK, 128), jnp.float32),
            pltpu.VMEM_SHARED((NP, 128), jnp.float32),
            pltpu.SemaphoreType.DMA,
            pltpu.SemaphoreType.DMA,
            pltpu.SemaphoreType.DMA,
            pltpu.SemaphoreType.DMA,
        ],
    )(functools.partial(_agg_body, col_split, ngrp, ng))


NGRP1 = 2
_agg1_kernel = _make_agg(True, NGRP1, NCH_B // NGRP1)
_agg2_kernel = _make_agg(False, 1, NCH_W)


def _dis(dA_ref, dB_ref):
    return lax.rsqrt(1.0 + dA_ref[...] + dB_ref[...])


def _mm1_body(x_ref, w1_ref, dA_ref, dB_ref, o_ref):
    y = jnp.dot(x_ref[...], w1_ref[...], preferred_element_type=jnp.float32)
    o_ref[0] = _dis(dA_ref, dB_ref) * y


_mm1 = pl.pallas_call(
    _mm1_body,
    grid=(2, NB),
    in_specs=[
        pl.BlockSpec((M_BLK, D_IN), lambda c, i: (i, 0)),
        pl.BlockSpec((D_IN, D_HID // 2), lambda c, i: (0, c)),
        pl.BlockSpec((M_BLK, 1), lambda c, i: (i, 0)),
        pl.BlockSpec((M_BLK, 1), lambda c, i: (i + NB, 0)),
    ],
    out_specs=pl.BlockSpec((1, M_BLK, D_HID // 2), lambda c, i: (c, i, 0)),
    out_shape=jax.ShapeDtypeStruct((2, NP, D_HID // 2), jnp.float32),
)


def _mm2_body(aA_ref, aB_ref, dA_ref, dB_ref, b1_ref, w2_ref, o_ref):
    dis = _dis(dA_ref, dB_ref)
    hA = jnp.maximum(dis * aA_ref[...] + b1_ref[0:1, : D_HID // 2], 0.0)
    hB = jnp.maximum(dis * aB_ref[...] + b1_ref[0:1, D_HID // 2:], 0.0)
    y = (jnp.dot(hA, w2_ref[: D_HID // 2], preferred_element_type=jnp.float32)
         + jnp.dot(hB, w2_ref[D_HID // 2:], preferred_element_type=jnp.float32))
    o_ref[...] = jnp.pad(dis * y, ((0, 0), (0, 128 - D_OUT)))


_mm2 = pl.pallas_call(
    _mm2_body,
    grid=(NB,),
    in_specs=[
        pl.BlockSpec((M_BLK, D_HID // 2), lambda i: (i, 0)),
        pl.BlockSpec((M_BLK, D_HID // 2), lambda i: (i + NB, 0)),
        pl.BlockSpec((M_BLK, 1), lambda i: (i, 0)),
        pl.BlockSpec((M_BLK, 1), lambda i: (i + NB, 0)),
        pl.BlockSpec((1, D_HID), lambda i: (0, 0)),
        pl.BlockSpec((D_HID, D_OUT), lambda i: (0, 0)),
    ],
    out_specs=pl.BlockSpec((M_BLK, 128), lambda i: (i, 0)),
    out_shape=jax.ShapeDtypeStruct((NP, 128), jnp.float32),
)


def _fin_body(aA_ref, aB_ref, y2_ref, dA_ref, dB_ref, b2_ref, o_ref):
    dis = _dis(dA_ref, dB_ref)
    agg = (aA_ref[:, :D_OUT] + aB_ref[:, :D_OUT] - y2_ref[:, :D_OUT])
    o_ref[...] = dis * agg + b2_ref[0:1, :]


_fin = pl.pallas_call(
    _fin_body,
    grid=(NB,),
    in_specs=[
        pl.BlockSpec((M_BLK, 128), lambda i: (i, 0)),
        pl.BlockSpec((M_BLK, 128), lambda i: (i + NB, 0)),
        pl.BlockSpec((M_BLK, 128), lambda i: (i, 0)),
        pl.BlockSpec((M_BLK, 1), lambda i: (i, 0)),
        pl.BlockSpec((M_BLK, 1), lambda i: (i + NB, 0)),
        pl.BlockSpec((1, D_OUT), lambda i: (0, 0)),
    ],
    out_specs=pl.BlockSpec((M_BLK, D_OUT), lambda i: (i, 0)),
    out_shape=jax.ShapeDtypeStruct((NP, D_OUT), jnp.float32),
)


def kernel(x, edge_index, W1, b1, W2, b2):
    src = edge_index[0].astype(jnp.int32)
    dst = edge_index[1].astype(jnp.int32)
    epad = jnp.full((EP - N_EDGES,), PAD, jnp.int32)
    srcp = jnp.concatenate([src, epad])
    dstp = jnp.concatenate([dst, epad])
    x_pad = jnp.concatenate(
        [x, jnp.zeros((NP - N_NODES, D_IN), jnp.float32)])

    dst_w = dstp.reshape(NSC * NTILE, NCH_W, CHUNK)
    src_w = srcp.reshape(NSC * NTILE, NCH_W, CHUNK)
    dst_f = dstp.reshape(EP // CHUNK, CHUNK)
    src_f = srcp.reshape(EP // CHUNK, CHUNK)

    zeros_slab = jnp.zeros((DEG_R, 128), jnp.float32)
    ident = jnp.arange(DEG_R, dtype=jnp.int32)

    deg = _deg_kernel(dst_deg, zeros_slab, ident)
    degr = deg.reshape(2 * NP, 1)
    y1 = _mm1(x_pad, W1, degr, degr)
    acc1 = _agg1_kernel(y1, src_b, dst_b)
    y2 = _mm2(acc1, acc1, degr, degr, b1.reshape(1, D_HID), W2)
    acc2 = _agg2_kernel(y2, src_f, dst_f)
    out = _fin(acc2, acc2, y2, degr, degr, b2.reshape(1, D_OUT))
    return out[:N_NODES]


# speedup vs baseline: 7.0128x; 1.0011x over previous
import functools

import jax
import jax.numpy as jnp
from jax import lax
from jax.experimental import pallas as pl
from jax.experimental.pallas import tpu as pltpu
from jax.experimental.pallas import tpu_sc as plsc

N_NODES = 10000
N_EDGES = 160000
D_IN = 256
D_HID = 256
D_OUT = 64

NP = 10240
EP = 163840
PAD = 10200
NSC = 2
NTILE = 16
ROWS_T = NP // NTILE
CHUNK = 128
NCH_B = EP // NTILE // CHUNK
NCH_W = EP // (NSC * NTILE) // CHUNK
EW = EP // (NSC * NTILE)
DEG_R = NP // 128
DEG_T = DEG_R // NTILE
M_BLK = 256
NB = NP // M_BLK
NG_SKEW = 16

_MESH = plsc.VectorSubcoreMesh(core_axis_name="c", subcore_axis_name="s",
                               num_cores=NSC, num_subcores=NTILE)


def _deg_body(dst_hbm, zeros_hbm, ident_hbm, out_hbm, idx_v, cnt, ident, acc):
    c = lax.axis_index("c")
    s = lax.axis_index("s")
    w = c * NTILE + s
    pltpu.sync_copy(dst_hbm.at[w], idx_v)
    pltpu.sync_copy(zeros_hbm, cnt)
    pltpu.sync_copy(ident_hbm, ident)
    @pl.when(s < DEG_R // 8)
    def _():
        pltpu.sync_copy(zeros_hbm.at[pl.ds(s * 8, 8)],
                        acc.at[pl.ds(s * 8, 8)])
    plsc.subcore_barrier()

    ones = jnp.full((16,), 1.0, jnp.float32)

    def body(j, carry):
        for l in range(8):
            v = idx_v[j, pl.ds(16 * l, 16)]
            row = jax.lax.shift_right_logical(v, 7)
            col = jax.lax.bitwise_and(v, 127)
            plsc.addupdate_scatter(cnt, [row, col], ones)
        return carry

    lax.fori_loop(0, EW // CHUNK, body, 0)
    pltpu.sync_copy(cnt, acc.at[ident], add=True)
    plsc.subcore_barrier()

    @pl.when(s < DEG_R // 8)
    def _():
        pltpu.sync_copy(acc.at[pl.ds(s * 8, 8)],
                        out_hbm.at[pl.ds(c * DEG_R + s * 8, 8)])


def _make_deg():
    return functools.partial(
        pl.kernel,
        out_type=jax.ShapeDtypeStruct((2 * DEG_R, 128), jnp.float32),
        mesh=_MESH,
        scratch_types=[
            pltpu.VMEM((NCH_W, CHUNK), jnp.int32),
            pltpu.VMEM((DEG_R, 128), jnp.float32),
            pltpu.VMEM((DEG_R,), jnp.int32),
            pltpu.VMEM_SHARED((DEG_R, 128), jnp.float32),
        ],
        compiler_params=pltpu.CompilerParams(needs_layout_passes=False),
    )(_deg_body)


_deg_kernel = _make_deg()


def _agg_body(col_split, ngrp, ng, y_hbm, src_hbm, dst_hbm, out_hbm,
              idx_s, idx_d, buf0, buf1, acc, sem0, sem1, sem_s0, sem_s1):
    c = lax.axis_index("c")
    s = lax.axis_index("s")
    w = c * NTILE + s
    if col_split:
        y_tab = y_hbm.at[c]
    else:
        y_tab = y_hbm
    pltpu.sync_copy(y_tab.at[pl.ds(s * ROWS_T, ROWS_T)],
                    acc.at[pl.ds(s * ROWS_T, ROWS_T)])
    plsc.subcore_barrier()

    def drain(g0, g1):
        pltpu.make_async_copy(buf0, acc.at[idx_d.at[g0]], sem_s0).wait()
        pltpu.make_async_copy(buf1, acc.at[idx_d.at[g1]], sem_s1).wait()

    def step(j2, carry):
        g = 2 * j2
        @pl.when(j2 > 0)
        def _():
            drain(g - 2, g - 1)
        d0 = pltpu.async_copy(y_tab.at[idx_s.at[g]], buf0, sem0)
        d1 = pltpu.async_copy(y_tab.at[idx_s.at[g + 1]], buf1, sem1)
        d0.wait()
        pltpu.async_copy(buf0, acc.at[idx_d.at[g]], sem_s0, add=True)
        d1.wait()
        pltpu.async_copy(buf1, acc.at[idx_d.at[g + 1]], sem_s1, add=True)
        return carry

    if col_split:
        for grp in range(ngrp):
            if grp > 0:
                drain(ng - 2, ng - 1)
            ibase = s * ngrp + grp
            pltpu.sync_copy(src_hbm.at[ibase], idx_s)
            pltpu.sync_copy(dst_hbm.at[ibase], idx_d)
            lax.fori_loop(0, ng // 2, step, 0)
    else:
        ng0, ng1 = ng - NG_SKEW, ng + NG_SKEW
        n_w = jnp.where(c == 0, ng0, ng1)
        start = pl.multiple_of(
            jnp.where(c == 0, s * ng0, NTILE * ng0 + s * ng1), 8)
        pltpu.sync_copy(src_hbm.at[pl.ds(start, ng1)], idx_s)
        pltpu.sync_copy(dst_hbm.at[pl.ds(start, ng1)], idx_d)
        lax.fori_loop(0, n_w // 2, step, 0)
    drain(ng - 2, ng - 1)
    plsc.subcore_barrier()
    pltpu.sync_copy(acc.at[pl.ds(s * ROWS_T, ROWS_T)],
                    out_hbm.at[pl.ds(c * NP + s * ROWS_T, ROWS_T)])


def _make_agg(col_split, ngrp, ng):
    return functools.partial(
        pl.kernel,
        out_type=jax.ShapeDtypeStruct((2 * NP, 128), jnp.float32),
        mesh=_MESH,
        scratch_types=[
            pltpu.VMEM((ng if col_split else ng + NG_SKEW, CHUNK), jnp.int32),
            pltpu.VMEM((ng if col_split else ng + NG_SKEW, CHUNK), jnp.int32),
            pltpu.VMEM((CHUNK, 128), jnp.float32),
            pltpu.VMEM((CHUNK, 128), jnp.float32),
            pltpu.VMEM_SHARED((NP, 128), jnp.float32),
            pltpu.SemaphoreType.DMA,
            pltpu.SemaphoreType.DMA,
            pltpu.SemaphoreType.DMA,
            pltpu.SemaphoreType.DMA,
        ],
    )(functools.partial(_agg_body, col_split, ngrp, ng))


NGRP1 = 2
_agg1_kernel = _make_agg(True, NGRP1, NCH_B // NGRP1)
_agg2_kernel = _make_agg(False, 1, NCH_W)


def _dis(dA_ref, dB_ref):
    return lax.rsqrt(1.0 + dA_ref[...] + dB_ref[...])


def _mm1_body(x_ref, w1_ref, dA_ref, dB_ref, o_ref):
    y = jnp.dot(x_ref[...], w1_ref[...], preferred_element_type=jnp.float32)
    o_ref[0] = _dis(dA_ref, dB_ref) * y


_mm1 = pl.pallas_call(
    _mm1_body,
    grid=(2, NB),
    in_specs=[
        pl.BlockSpec((M_BLK, D_IN), lambda c, i: (i, 0)),
        pl.BlockSpec((D_IN, D_HID // 2), lambda c, i: (0, c)),
        pl.BlockSpec((M_BLK, 1), lambda c, i: (i, 0)),
        pl.BlockSpec((M_BLK, 1), lambda c, i: (i + NB, 0)),
    ],
    out_specs=pl.BlockSpec((1, M_BLK, D_HID // 2), lambda c, i: (c, i, 0)),
    out_shape=jax.ShapeDtypeStruct((2, NP, D_HID // 2), jnp.float32),
)


def _mm2_body(aA_ref, aB_ref, dA_ref, dB_ref, b1_ref, w2_ref, o_ref):
    dis = _dis(dA_ref, dB_ref)
    hA = jnp.maximum(dis * aA_ref[...] + b1_ref[0:1, : D_HID // 2], 0.0)
    hB = jnp.maximum(dis * aB_ref[...] + b1_ref[0:1, D_HID // 2:], 0.0)
    y = (jnp.dot(hA, w2_ref[: D_HID // 2], preferred_element_type=jnp.float32)
         + jnp.dot(hB, w2_ref[D_HID // 2:], preferred_element_type=jnp.float32))
    o_ref[...] = jnp.pad(dis * y, ((0, 0), (0, 128 - D_OUT)))


_mm2 = pl.pallas_call(
    _mm2_body,
    grid=(NB,),
    in_specs=[
        pl.BlockSpec((M_BLK, D_HID // 2), lambda i: (i, 0)),
        pl.BlockSpec((M_BLK, D_HID // 2), lambda i: (i + NB, 0)),
        pl.BlockSpec((M_BLK, 1), lambda i: (i, 0)),
        pl.BlockSpec((M_BLK, 1), lambda i: (i + NB, 0)),
        pl.BlockSpec((1, D_HID), lambda i: (0, 0)),
        pl.BlockSpec((D_HID, D_OUT), lambda i: (0, 0)),
    ],
    out_specs=pl.BlockSpec((M_BLK, 128), lambda i: (i, 0)),
    out_shape=jax.ShapeDtypeStruct((NP, 128), jnp.float32),
)


def _fin_body(aA_ref, aB_ref, y2_ref, dA_ref, dB_ref, b2_ref, o_ref):
    dis = _dis(dA_ref, dB_ref)
    agg = (aA_ref[:, :D_OUT] + aB_ref[:, :D_OUT] - y2_ref[:, :D_OUT])
    o_ref[...] = dis * agg + b2_ref[0:1, :]


_fin = pl.pallas_call(
    _fin_body,
    grid=(NB,),
    in_specs=[
        pl.BlockSpec((M_BLK, 128), lambda i: (i, 0)),
        pl.BlockSpec((M_BLK, 128), lambda i: (i + NB, 0)),
        pl.BlockSpec((M_BLK, 128), lambda i: (i, 0)),
        pl.BlockSpec((M_BLK, 1), lambda i: (i, 0)),
        pl.BlockSpec((M_BLK, 1), lambda i: (i + NB, 0)),
        pl.BlockSpec((1, D_OUT), lambda i: (0, 0)),
    ],
    out_specs=pl.BlockSpec((M_BLK, D_OUT), lambda i: (i, 0)),
    out_shape=jax.ShapeDtypeStruct((NP, D_OUT), jnp.float32),
)


def kernel(x, edge_index, W1, b1, W2, b2):
    src = edge_index[0].astype(jnp.int32)
    dst = edge_index[1].astype(jnp.int32)
    epad = jnp.full((EP - N_EDGES,), PAD, jnp.int32)
    srcp = jnp.concatenate([src, epad])
    dstp = jnp.concatenate([dst, epad])
    x_pad = jnp.concatenate(
        [x, jnp.zeros((NP - N_NODES, D_IN), jnp.float32)])

    dst_w = dstp.reshape(NSC * NTILE, NCH_W, CHUNK)
    src_w = srcp.reshape(NSC * NTILE, NCH_W, CHUNK)
    dst_f = dstp.reshape(EP // CHUNK, CHUNK)
    src_f = srcp.reshape(EP // CHUNK, CHUNK)

    zeros_slab = jnp.zeros((DEG_R, 128), jnp.float32)
    ident = jnp.arange(DEG_R, dtype=jnp.int32)

    deg = _deg_kernel(dst_w, zeros_slab, ident)
    degr = deg.reshape(2 * NP, 1)
    y1 = _mm1(x_pad, W1, degr, degr)
    acc1 = _agg1_kernel(y1, src_w, dst_w)
    y2 = _mm2(acc1, acc1, degr, degr, b1.reshape(1, D_HID), W2)
    acc2 = _agg2_kernel(y2, src_f, dst_f)
    out = _fin(acc2, acc2, y2, degr, degr, b2.reshape(1, D_OUT))
    return out[:N_NODES]
